# baseline (speedup 1.0000x reference)
import functools

import jax
import jax.numpy as jnp
from jax import lax
from jax.experimental import pallas as pl
from jax.experimental.pallas import tpu as pltpu

E = 8
K = 2
D = 768
FF = 1024
T = 2048

_BT = 256


def _router_body(x_ref, rw_ref, comb_ref):
    xb = x_ref[...]
    logits = lax.dot_general(xb, rw_ref[...], (((1,), (1,)), ((), ())),
                             preferred_element_type=jnp.float32)
    idx = lax.broadcasted_iota(jnp.int32, logits.shape, 1)
    l1 = jnp.max(logits, axis=-1, keepdims=True)
    i1 = jnp.min(jnp.where(logits == l1, idx, E), axis=-1, keepdims=True)
    m1 = idx == i1
    masked = jnp.where(m1, -jnp.inf, logits)
    l2 = jnp.max(masked, axis=-1, keepdims=True)
    i2 = jnp.min(jnp.where(masked == l2, idx, E), axis=-1, keepdims=True)
    m2 = idx == i2
    w1 = jax.nn.sigmoid(l1 - l2)
    comb_ref[...] = jnp.where(m1, w1, 0.0) + jnp.where(m2, 1.0 - w1, 0.0)


def _moe_body(comb_ref, x_ref, gu_ref, dp_ref, out_ref):
    e = pl.program_id(1)
    xb = x_ref[...]
    gu = lax.dot_general(xb, gu_ref[0], (((1,), (1,)), ((), ())),
                         preferred_element_type=jnp.float32)
    gate = gu[:, :FF]
    up = gu[:, FF:]
    h = gate * jax.nn.sigmoid(gate) * up
    y = lax.dot_general(h, dp_ref[0], (((1,), (1,)), ((), ())),
                        preferred_element_type=jnp.float32)
    eidx = lax.broadcasted_iota(jnp.int32, (comb_ref.shape[0], E), 1)
    w = jnp.sum(jnp.where(eidx == e, comb_ref[...], 0.0), axis=-1,
                keepdims=True)
    contrib = w * y

    @pl.when(e == 0)
    def _():
        out_ref[...] = contrib

    @pl.when(e > 0)
    def _():
        out_ref[...] = out_ref[...] + contrib


@functools.partial(jax.jit, static_argnames=("interpret",))
def kernel(x, router_weight, gate_up_proj, down_proj, interpret=False):
    Bb, Ss, Dd = x.shape
    xf = x.reshape(-1, Dd)

    comb = pl.pallas_call(
        _router_body,
        grid=(T // _BT,),
        in_specs=[
            pl.BlockSpec((_BT, D), lambda i: (i, 0)),
            pl.BlockSpec((E, D), lambda i: (0, 0)),
        ],
        out_specs=pl.BlockSpec((_BT, E), lambda i: (i, 0)),
        out_shape=jax.ShapeDtypeStruct((T, E), jnp.float32),
        interpret=interpret,
    )(xf, router_weight)

    out = pl.pallas_call(
        _moe_body,
        grid=(T // _BT, E),
        in_specs=[
            pl.BlockSpec((_BT, E), lambda i, e: (i, 0)),
            pl.BlockSpec((_BT, D), lambda i, e: (i, 0)),
            pl.BlockSpec((1, 2 * FF, D), lambda i, e: (e, 0, 0)),
            pl.BlockSpec((1, D, FF), lambda i, e: (e, 0, 0)),
        ],
        out_specs=pl.BlockSpec((_BT, D), lambda i, e: (i, 0)),
        out_shape=jax.ShapeDtypeStruct((T, D), jnp.float32),
        interpret=interpret,
    )(comb, xf, gate_up_proj, down_proj)

    return out.reshape(Bb, Ss, Dd)
